# baseline (speedup 1.0000x reference)
import functools
import jax
import jax.numpy as jnp
from jax import lax
from jax.experimental import pallas as pl
from jax.experimental.pallas import tpu as pltpu
from jax.experimental.pallas import tpu_sc as plsc

_SPATIAL = ((64, 64), (32, 32), (16, 16), (8, 8))
_LVL_BASE = (0, 4096, 5120, 5376)
_EMBED = 256
_NL = 4
_NH = 8
_NP = 4
_BS = 4
_LQ = 1024
_LV = 5440
_C = 32
_NCORN = _NL * _NP * 4

_TV = 680


def _vproj_body(val_ref, msk_ref, w_ref, b_ref, out_ref):
    x = val_ref[0]
    v = jnp.dot(x, w_ref[...], preferred_element_type=jnp.float32)
    v = (v + b_ref[...]) * msk_ref[0, 0, 0][:, None]
    for h in range(_NH):
        out_ref[0, h] = v[:, h * _C:(h + 1) * _C]


def _vproj(value, maskf, W_value, b_value):
    return pl.pallas_call(
        _vproj_body,
        grid=(_BS, _LV // _TV),
        in_specs=[
            pl.BlockSpec((1, _TV, _EMBED), lambda b, t: (b, t, 0)),
            pl.BlockSpec((1, 1, 1, _TV), lambda b, t: (b, t, 0, 0)),
            pl.BlockSpec((_EMBED, _EMBED), lambda b, t: (0, 0)),
            pl.BlockSpec((1, _EMBED), lambda b, t: (0, 0)),
        ],
        out_specs=pl.BlockSpec((1, _NH, _TV, _C), lambda b, t: (b, 0, t, 0)),
        out_shape=jax.ShapeDtypeStruct((_BS, _NH, _LV, _C), jnp.float32),
    )(value, maskf, W_value, b_value)


_QT = 256


def _prep_body(q_ref, rp_ref, wox_ref, box_ref, woy_ref, boy_ref,
               wat_ref, bat_ref, idx_ref, wts_ref):
    b = pl.program_id(0)
    q = q_ref[0]
    offx = jnp.dot(q, wox_ref[...], preferred_element_type=jnp.float32) + box_ref[...]
    offy = jnp.dot(q, woy_ref[...], preferred_element_type=jnp.float32) + boy_ref[...]
    logits = jnp.dot(q, wat_ref[...], preferred_element_type=jnp.float32) + bat_ref[...]

    el = jnp.exp(logits)
    hr = lax.broadcasted_iota(jnp.int32, (128, 128), 0) >> 4
    hc = lax.broadcasted_iota(jnp.int32, (128, 128), 1) >> 4
    seg = (hr == hc).astype(jnp.float32)
    aw = el / jnp.dot(el, seg, preferred_element_type=jnp.float32,
                      precision=lax.Precision.HIGHEST)

    col = lax.broadcasted_iota(jnp.int32, (_QT, 128), 1)
    lvl = (col >> 2) & 3
    wvec_i = jnp.right_shift(jnp.int32(64), lvl)
    wvec = wvec_i.astype(jnp.float32)
    lb = (16384 - jnp.right_shift(jnp.int32(16384), 2 * lvl)) // 3

    rowi = lax.broadcasted_iota(jnp.int32, (8, 128), 0)
    lvlj = (lax.broadcasted_iota(jnp.int32, (8, 128), 1) >> 2) & 3
    selx = (rowi == lvlj).astype(jnp.float32)
    sely = (rowi == lvlj + 4).astype(jnp.float32)
    rp = rp_ref[0]
    rxw = jnp.dot(rp, selx * wvec[:1], preferred_element_type=jnp.float32,
                  precision=lax.Precision.HIGHEST)
    ryw = jnp.dot(rp, sely * wvec[:1], preferred_element_type=jnp.float32,
                  precision=lax.Precision.HIGHEST)

    x = rxw + offx - 0.5
    y = ryw + offy - 0.5
    x0f = jnp.floor(x)
    y0f = jnp.floor(y)
    fx = x - x0f
    fy = y - y0f
    x0in = (x0f >= 0.0) & (x0f <= wvec - 1.0)
    x1in = (x0f + 1.0 >= 0.0) & (x0f + 1.0 <= wvec - 1.0)
    y0in = (y0f >= 0.0) & (y0f <= wvec - 1.0)
    y1in = (y0f + 1.0 >= 0.0) & (y0f + 1.0 <= wvec - 1.0)
    x0c = jnp.clip(x0f, 0.0, wvec - 1.0).astype(jnp.int32)
    x1c = jnp.clip(x0f + 1.0, 0.0, wvec - 1.0).astype(jnp.int32)
    y0c = jnp.clip(y0f, 0.0, wvec - 1.0).astype(jnp.int32)
    y1c = jnp.clip(y0f + 1.0, 0.0, wvec - 1.0).astype(jnp.int32)
    gb = lb + (b * _NH + (col >> 4)) * _LV
    r0 = gb + y0c * wvec_i
    r1 = gb + y1c * wvec_i
    idx_ref[0, 0] = r0 + x0c
    idx_ref[0, 1] = r1 + x0c
    idx_ref[0, 2] = r0 + x1c
    idx_ref[0, 3] = r1 + x1c
    gx = 1.0 - fx
    gy = 1.0 - fy
    wts_ref[0, 0] = jnp.where(x0in & y0in, gx * gy, 0.0) * aw
    wts_ref[0, 1] = jnp.where(x0in & y1in, gx * fy, 0.0) * aw
    wts_ref[0, 2] = jnp.where(x1in & y0in, fx * gy, 0.0) * aw
    wts_ref[0, 3] = jnp.where(x1in & y1in, fx * fy, 0.0) * aw


def _prep(query, rp, Wox, box, Woy, boy, W_attn, b_attn):
    return pl.pallas_call(
        _prep_body,
        grid=(_BS, _LQ // _QT),
        in_specs=[
            pl.BlockSpec((1, _QT, _EMBED), lambda b, t: (b, t, 0)),
            pl.BlockSpec((1, _QT, 2 * _NL), lambda b, t: (b, t, 0)),
            pl.BlockSpec((_EMBED, 128), lambda b, t: (0, 0)),
            pl.BlockSpec((1, 128), lambda b, t: (0, 0)),
            pl.BlockSpec((_EMBED, 128), lambda b, t: (0, 0)),
            pl.BlockSpec((1, 128), lambda b, t: (0, 0)),
            pl.BlockSpec((_EMBED, 128), lambda b, t: (0, 0)),
            pl.BlockSpec((1, 128), lambda b, t: (0, 0)),
        ],
        out_specs=[
            pl.BlockSpec((1, 4, _QT, 128), lambda b, t: (b, 0, t, 0)),
            pl.BlockSpec((1, 4, _QT, 128), lambda b, t: (b, 0, t, 0)),
        ],
        out_shape=[
            jax.ShapeDtypeStruct((_BS, 4, _LQ, 128), jnp.int32),
            jax.ShapeDtypeStruct((_BS, 4, _LQ, 128), jnp.float32),
        ],
    )(query, rp, Wox, box, Woy, boy, W_attn, b_attn)


_CQ = 16
_NCHUNK = _LQ // _CQ
_NROW = _CQ * _NCORN


_NB = _NROW // 128


def _sc_sample(table, iwh):
    info = plsc.get_sparse_core_info()
    nc = info.num_cores
    mesh = plsc.VectorSubcoreMesh(core_axis_name="c", subcore_axis_name="s")

    @functools.partial(
        pl.kernel,
        out_type=jax.ShapeDtypeStruct((_BS * _NH, _LQ, _C), jnp.float32),
        mesh=mesh,
        compiler_params=pltpu.CompilerParams(needs_layout_passes=False,
                                             use_tc_tiling_on_sc=False),
        scratch_types=[
            pltpu.VMEM((2 * _NB, 128), jnp.int32),
            pltpu.VMEM((2 * _NB, 128), jnp.int32),
            pltpu.VMEM((_NROW, _C), jnp.float32),
            pltpu.VMEM((_NROW, _C), jnp.float32),
            pltpu.VMEM((_CQ, _C), jnp.float32),
            pltpu.SemaphoreType.DMA,
            pltpu.SemaphoreType.DMA,
        ],
    )
    def run(table_h, iw_h, out_h, iw0, iw1, rows0, rows1, out_v, gs0, gs1):
        wid = lax.axis_index("s") * nc + lax.axis_index("c")
        col0 = lax.iota(jnp.int32, 16)
        col1 = col0 + 16

        def load_iw(n, iwv):
            pltpu.sync_copy(iw_h.at[wid, n], iwv)

        def gather(iwv, rowsv, sem):
            for s in range(_NB):
                pltpu.async_copy(table_h.at[iwv.at[s]],
                                 rowsv.at[pl.ds(s * 128, 128)], sem)

        def drain(rowsv, sem):
            pltpu.make_async_copy(table_h.at[pl.ds(0, _NROW)], rowsv, sem).wait()

        def compute(iwv, rowsv, n):
            def qloop(qi, c2):
                base = jnp.full((16,), qi * _CQ, jnp.int32)
                acc0 = None
                acc1 = None
                for c in range(4):
                    for lp in range(16):
                        rsp = base + (c * 256 + lp)
                        wi = plsc.load_gather(iwv, [(rsp >> 7) + _NB, rsp & 127])
                        w = plsc.bitcast(wi, jnp.float32)
                        r0 = plsc.load_gather(rowsv, [rsp, col0])
                        r1 = plsc.load_gather(rowsv, [rsp, col1])
                        if acc0 is None:
                            acc0 = w * r0
                            acc1 = w * r1
                        else:
                            acc0 = acc0 + w * r0
                            acc1 = acc1 + w * r1
                qsp = jnp.full((16,), qi, jnp.int32)
                plsc.store_scatter(out_v, [qsp, col0], acc0)
                plsc.store_scatter(out_v, [qsp, col1], acc1)
                return c2

            lax.fori_loop(0, _CQ, qloop, 0)
            pltpu.sync_copy(out_v, out_h.at[wid, pl.ds(n * _CQ, _CQ)])

        load_iw(0, iw0)
        gather(iw0, rows0, gs0)

        def body(i, carry):
            n0 = 2 * i
            load_iw(n0 + 1, iw1)
            gather(iw1, rows1, gs1)
            drain(rows0, gs0)
            compute(iw0, rows0, n0)

            @pl.when(i < _NCHUNK // 2 - 1)
            def _():
                load_iw(n0 + 2, iw0)
                gather(iw0, rows0, gs0)

            drain(rows1, gs1)
            compute(iw1, rows1, n0 + 1)
            return carry

        lax.fori_loop(0, _NCHUNK // 2, body, 0)

    return run(table, iwh)


_QTC = 512


def _outproj_body(s_ref, w_ref, b_ref, o_ref):
    parts = [s_ref[0, h] for h in range(_NH)]
    x = jnp.concatenate(parts, axis=1)
    o_ref[0] = jnp.dot(x, w_ref[...], preferred_element_type=jnp.float32) + b_ref[...]


def _outproj(sampled, W_out, b_out):
    return pl.pallas_call(
        _outproj_body,
        grid=(_BS, _LQ // _QTC),
        in_specs=[
            pl.BlockSpec((1, _NH, _QTC, _C), lambda b, t: (b, 0, t, 0)),
            pl.BlockSpec((_EMBED, _EMBED), lambda b, t: (0, 0)),
            pl.BlockSpec((1, _EMBED), lambda b, t: (0, 0)),
        ],
        out_specs=pl.BlockSpec((1, _QTC, _EMBED), lambda b, t: (b, t, 0)),
        out_shape=jax.ShapeDtypeStruct((_BS, _LQ, _EMBED), jnp.float32),
    )(sampled, W_out, b_out)


def kernel(query, ref_points, value, pad_mask, W_value, b_value, W_off, b_off,
           W_attn, b_attn, W_out, b_out):
    maskf = pad_mask.astype(jnp.float32).reshape(_BS, _LV // _TV, 1, _TV)
    table = _vproj(value, maskf, W_value, b_value.reshape(1, _EMBED))
    table = table.reshape(_BS * _NH * _LV, _C)

    Wo = W_off.reshape(_EMBED, _NH * _NL * _NP, 2)
    bo = b_off.reshape(_NH * _NL * _NP, 2)
    rp = jnp.concatenate([ref_points[..., 0], ref_points[..., 1]], axis=-1)
    idx, wts = _prep(query, rp,
                     Wo[..., 0], bo[:, 0].reshape(1, -1),
                     Wo[..., 1], bo[:, 1].reshape(1, -1),
                     W_attn, b_attn.reshape(1, -1))

    def _to_worker(a):
        a = a.reshape(_BS, 4, _NCHUNK, _CQ, _NH, 16)
        a = a.transpose(0, 4, 2, 1, 3, 5)
        return a.reshape(_BS * _NH, _NCHUNK, _NB, 128)

    idxh = _to_worker(idx)
    wtsh = _to_worker(jax.lax.bitcast_convert_type(wts, jnp.int32))
    iwh = jnp.concatenate([idxh, wtsh], axis=2)
    sampled = _sc_sample(table, iwh)
    sampled = sampled.reshape(_BS, _NH, _LQ, _C)
    return _outproj(sampled, W_out, b_out.reshape(1, _EMBED))


# speedup vs baseline: 1.0178x; 1.0178x over previous
import functools
import jax
import jax.numpy as jnp
from jax import lax
from jax.experimental import pallas as pl
from jax.experimental.pallas import tpu as pltpu
from jax.experimental.pallas import tpu_sc as plsc

_SPATIAL = ((64, 64), (32, 32), (16, 16), (8, 8))
_LVL_BASE = (0, 4096, 5120, 5376)
_EMBED = 256
_NL = 4
_NH = 8
_NP = 4
_BS = 4
_LQ = 1024
_LV = 5440
_C = 32
_NCORN = _NL * _NP * 4

_TV = 680


def _vproj_body(val_ref, msk_ref, w_ref, b_ref, out_ref):
    x = val_ref[0]
    v = jnp.dot(x, w_ref[...], preferred_element_type=jnp.float32)
    v = ((v + b_ref[...]) * msk_ref[0, 0, 0][:, None]).astype(jnp.bfloat16)
    for h in range(_NH):
        out_ref[0, h] = v[:, h * _C:(h + 1) * _C]


def _vproj(value, maskf, W_value, b_value):
    return pl.pallas_call(
        _vproj_body,
        grid=(_BS, _LV // _TV),
        in_specs=[
            pl.BlockSpec((1, _TV, _EMBED), lambda b, t: (b, t, 0)),
            pl.BlockSpec((1, 1, 1, _TV), lambda b, t: (b, t, 0, 0)),
            pl.BlockSpec((_EMBED, _EMBED), lambda b, t: (0, 0)),
            pl.BlockSpec((1, _EMBED), lambda b, t: (0, 0)),
        ],
        out_specs=pl.BlockSpec((1, _NH, _TV, _C), lambda b, t: (b, 0, t, 0)),
        out_shape=jax.ShapeDtypeStruct((_BS, _NH, _LV, _C), jnp.bfloat16),
    )(value, maskf, W_value, b_value)


_QT = 256


def _prep_body(q_ref, rp_ref, wox_ref, box_ref, woy_ref, boy_ref,
               wat_ref, bat_ref, idx_ref, wts_ref):
    b = pl.program_id(0)
    q = q_ref[0]
    offx = jnp.dot(q, wox_ref[...], preferred_element_type=jnp.float32) + box_ref[...]
    offy = jnp.dot(q, woy_ref[...], preferred_element_type=jnp.float32) + boy_ref[...]
    logits = jnp.dot(q, wat_ref[...], preferred_element_type=jnp.float32) + bat_ref[...]

    el = jnp.exp(logits)
    hr = lax.broadcasted_iota(jnp.int32, (128, 128), 0) >> 4
    hc = lax.broadcasted_iota(jnp.int32, (128, 128), 1) >> 4
    seg = (hr == hc).astype(jnp.float32)
    aw = el / jnp.dot(el, seg, preferred_element_type=jnp.float32,
                      precision=lax.Precision.HIGHEST)

    col = lax.broadcasted_iota(jnp.int32, (_QT, 128), 1)
    lvl = (col >> 2) & 3
    wvec_i = jnp.right_shift(jnp.int32(64), lvl)
    wvec = wvec_i.astype(jnp.float32)
    lb = (16384 - jnp.right_shift(jnp.int32(16384), 2 * lvl)) // 3

    rowi = lax.broadcasted_iota(jnp.int32, (8, 128), 0)
    lvlj = (lax.broadcasted_iota(jnp.int32, (8, 128), 1) >> 2) & 3
    selx = (rowi == lvlj).astype(jnp.float32)
    sely = (rowi == lvlj + 4).astype(jnp.float32)
    rp = rp_ref[0]
    rxw = jnp.dot(rp, selx * wvec[:1], preferred_element_type=jnp.float32,
                  precision=lax.Precision.HIGHEST)
    ryw = jnp.dot(rp, sely * wvec[:1], preferred_element_type=jnp.float32,
                  precision=lax.Precision.HIGHEST)

    x = rxw + offx - 0.5
    y = ryw + offy - 0.5
    x0f = jnp.floor(x)
    y0f = jnp.floor(y)
    fx = x - x0f
    fy = y - y0f
    x0in = (x0f >= 0.0) & (x0f <= wvec - 1.0)
    x1in = (x0f + 1.0 >= 0.0) & (x0f + 1.0 <= wvec - 1.0)
    y0in = (y0f >= 0.0) & (y0f <= wvec - 1.0)
    y1in = (y0f + 1.0 >= 0.0) & (y0f + 1.0 <= wvec - 1.0)
    x0c = jnp.clip(x0f, 0.0, wvec - 1.0).astype(jnp.int32)
    x1c = jnp.clip(x0f + 1.0, 0.0, wvec - 1.0).astype(jnp.int32)
    y0c = jnp.clip(y0f, 0.0, wvec - 1.0).astype(jnp.int32)
    y1c = jnp.clip(y0f + 1.0, 0.0, wvec - 1.0).astype(jnp.int32)
    gb = lb + (b * _NH + (col >> 4)) * _LV
    r0 = gb + y0c * wvec_i
    r1 = gb + y1c * wvec_i
    idx_ref[0, 0] = r0 + x0c
    idx_ref[0, 1] = r1 + x0c
    idx_ref[0, 2] = r0 + x1c
    idx_ref[0, 3] = r1 + x1c
    gx = 1.0 - fx
    gy = 1.0 - fy
    wts_ref[0, 0] = jnp.where(x0in & y0in, gx * gy, 0.0) * aw
    wts_ref[0, 1] = jnp.where(x0in & y1in, gx * fy, 0.0) * aw
    wts_ref[0, 2] = jnp.where(x1in & y0in, fx * gy, 0.0) * aw
    wts_ref[0, 3] = jnp.where(x1in & y1in, fx * fy, 0.0) * aw


def _prep(query, rp, Wox, box, Woy, boy, W_attn, b_attn):
    return pl.pallas_call(
        _prep_body,
        grid=(_BS, _LQ // _QT),
        in_specs=[
            pl.BlockSpec((1, _QT, _EMBED), lambda b, t: (b, t, 0)),
            pl.BlockSpec((1, _QT, 2 * _NL), lambda b, t: (b, t, 0)),
            pl.BlockSpec((_EMBED, 128), lambda b, t: (0, 0)),
            pl.BlockSpec((1, 128), lambda b, t: (0, 0)),
            pl.BlockSpec((_EMBED, 128), lambda b, t: (0, 0)),
            pl.BlockSpec((1, 128), lambda b, t: (0, 0)),
            pl.BlockSpec((_EMBED, 128), lambda b, t: (0, 0)),
            pl.BlockSpec((1, 128), lambda b, t: (0, 0)),
        ],
        out_specs=[
            pl.BlockSpec((1, 4, _QT, 128), lambda b, t: (b, 0, t, 0)),
            pl.BlockSpec((1, 4, _QT, 128), lambda b, t: (b, 0, t, 0)),
        ],
        out_shape=[
            jax.ShapeDtypeStruct((_BS, 4, _LQ, 128), jnp.int32),
            jax.ShapeDtypeStruct((_BS, 4, _LQ, 128), jnp.float32),
        ],
    )(query, rp, Wox, box, Woy, boy, W_attn, b_attn)


_CQ = 16
_NCHUNK = _LQ // _CQ
_NROW = _CQ * _NCORN


_NB = _NROW // 128


def _sc_sample(table, iwh):
    info = plsc.get_sparse_core_info()
    nc = info.num_cores
    mesh = plsc.VectorSubcoreMesh(core_axis_name="c", subcore_axis_name="s")

    @functools.partial(
        pl.kernel,
        out_type=jax.ShapeDtypeStruct((_BS * _NH, _LQ, _C), jnp.float32),
        mesh=mesh,
        compiler_params=pltpu.CompilerParams(needs_layout_passes=False,
                                             use_tc_tiling_on_sc=False),
        scratch_types=[
            pltpu.VMEM((2 * _NB, 128), jnp.int32),
            pltpu.VMEM((2 * _NB, 128), jnp.int32),
            pltpu.VMEM((_NROW, _C // 2), jnp.int32),
            pltpu.VMEM((_NROW, _C // 2), jnp.int32),
            pltpu.VMEM((_CQ, _C), jnp.float32),
            pltpu.SemaphoreType.DMA,
            pltpu.SemaphoreType.DMA,
        ],
    )
    def run(table_h, iw_h, out_h, iw0, iw1, rows0, rows1, out_v, gs0, gs1):
        wid = lax.axis_index("s") * nc + lax.axis_index("c")
        col0 = lax.iota(jnp.int32, 16)
        col1 = col0 + 16

        def load_iw(n, iwv):
            pltpu.sync_copy(iw_h.at[wid, n], iwv)

        def gather(iwv, rowsv, sem):
            for s in range(_NB):
                pltpu.async_copy(table_h.at[iwv.at[s]],
                                 rowsv.at[pl.ds(s * 128, 128)], sem)

        def drain(rowsv, sem):
            pltpu.make_async_copy(table_h.at[pl.ds(0, _NROW)], rowsv, sem).wait()

        def compute(iwv, rowsv, n):
            def qloop(qi, c2):
                base = jnp.full((16,), qi * _CQ, jnp.int32)
                acc0 = None
                acc1 = None
                for c in range(4):
                    for lp in range(16):
                        rsp = base + (c * 256 + lp)
                        wi = plsc.load_gather(iwv, [(rsp >> 7) + _NB, rsp & 127])
                        w = plsc.bitcast(wi, jnp.float32)
                        ri = plsc.load_gather(rowsv, [rsp, col0])
                        bf = plsc.bitcast(ri, jnp.bfloat16)
                        re, ro = plsc.unpack(bf, format=plsc.PackFormat.INTERLEAVED)
                        if acc0 is None:
                            acc0 = w * re
                            acc1 = w * ro
                        else:
                            acc0 = acc0 + w * re
                            acc1 = acc1 + w * ro
                qsp = jnp.full((16,), qi, jnp.int32)
                plsc.store_scatter(out_v, [qsp, col0 * 2], acc0)
                plsc.store_scatter(out_v, [qsp, col0 * 2 + 1], acc1)
                return c2

            lax.fori_loop(0, _CQ, qloop, 0)
            pltpu.sync_copy(out_v, out_h.at[wid, pl.ds(n * _CQ, _CQ)])

        load_iw(0, iw0)
        gather(iw0, rows0, gs0)

        def body(i, carry):
            n0 = 2 * i
            load_iw(n0 + 1, iw1)
            gather(iw1, rows1, gs1)
            drain(rows0, gs0)
            compute(iw0, rows0, n0)

            @pl.when(i < _NCHUNK // 2 - 1)
            def _():
                load_iw(n0 + 2, iw0)
                gather(iw0, rows0, gs0)

            drain(rows1, gs1)
            compute(iw1, rows1, n0 + 1)
            return carry

        lax.fori_loop(0, _NCHUNK // 2, body, 0)

    return run(table, iwh)


_QTC = 512


def _outproj_body(s_ref, w_ref, b_ref, o_ref):
    parts = [s_ref[0, h] for h in range(_NH)]
    x = jnp.concatenate(parts, axis=1)
    o_ref[0] = jnp.dot(x, w_ref[...], preferred_element_type=jnp.float32) + b_ref[...]


def _outproj(sampled, W_out, b_out):
    return pl.pallas_call(
        _outproj_body,
        grid=(_BS, _LQ // _QTC),
        in_specs=[
            pl.BlockSpec((1, _NH, _QTC, _C), lambda b, t: (b, 0, t, 0)),
            pl.BlockSpec((_EMBED, _EMBED), lambda b, t: (0, 0)),
            pl.BlockSpec((1, _EMBED), lambda b, t: (0, 0)),
        ],
        out_specs=pl.BlockSpec((1, _QTC, _EMBED), lambda b, t: (b, t, 0)),
        out_shape=jax.ShapeDtypeStruct((_BS, _LQ, _EMBED), jnp.float32),
    )(sampled, W_out, b_out)


def kernel(query, ref_points, value, pad_mask, W_value, b_value, W_off, b_off,
           W_attn, b_attn, W_out, b_out):
    maskf = pad_mask.astype(jnp.float32).reshape(_BS, _LV // _TV, 1, _TV)
    table = _vproj(value, maskf, W_value, b_value.reshape(1, _EMBED))
    table = jax.lax.bitcast_convert_type(
        table.reshape(_BS * _NH * _LV, _C // 2, 2), jnp.int32)

    Wo = W_off.reshape(_EMBED, _NH * _NL * _NP, 2)
    bo = b_off.reshape(_NH * _NL * _NP, 2)
    rp = jnp.concatenate([ref_points[..., 0], ref_points[..., 1]], axis=-1)
    idx, wts = _prep(query, rp,
                     Wo[..., 0], bo[:, 0].reshape(1, -1),
                     Wo[..., 1], bo[:, 1].reshape(1, -1),
                     W_attn, b_attn.reshape(1, -1))

    def _to_worker(a):
        a = a.reshape(_BS, 4, _NCHUNK, _CQ, _NH, 16)
        a = a.transpose(0, 4, 2, 1, 3, 5)
        return a.reshape(_BS * _NH, _NCHUNK, _NB, 128)

    idxh = _to_worker(idx)
    wtsh = _to_worker(jax.lax.bitcast_convert_type(wts, jnp.int32))
    iwh = jnp.concatenate([idxh, wtsh], axis=2)
    sampled = _sc_sample(table, iwh)
    sampled = sampled.reshape(_BS, _NH, _LQ, _C)
    return _outproj(sampled, W_out, b_out.reshape(1, _EMBED))


# speedup vs baseline: 1.4576x; 1.4321x over previous
import functools
import jax
import jax.numpy as jnp
from jax import lax
from jax.experimental import pallas as pl
from jax.experimental.pallas import tpu as pltpu
from jax.experimental.pallas import tpu_sc as plsc

_SPATIAL = ((64, 64), (32, 32), (16, 16), (8, 8))
_LVL_BASE = (0, 4096, 5120, 5376)
_EMBED = 256
_NL = 4
_NH = 8
_NP = 4
_BS = 4
_LQ = 1024
_LV = 5440
_C = 32
_NCORN = _NL * _NP * 4

_TV = 680


def _vproj_body(val_ref, msk_ref, w_ref, b_ref, out_ref):
    x = val_ref[0]
    v = jnp.dot(x, w_ref[...], preferred_element_type=jnp.float32)
    v = ((v + b_ref[...]) * msk_ref[0, 0, 0][:, None]).astype(jnp.bfloat16)
    for h in range(_NH):
        lo = jax.lax.bitcast_convert_type(
            v[:, h * _C:h * _C + 16], jnp.int16).astype(jnp.int32) & 0xFFFF
        hi = jax.lax.bitcast_convert_type(
            v[:, h * _C + 16:(h + 1) * _C], jnp.int16).astype(jnp.int32)
        out_ref[0, h] = lo | (hi << 16)


def _vproj(value, maskf, W_value, b_value):
    return pl.pallas_call(
        _vproj_body,
        grid=(_BS, _LV // _TV),
        in_specs=[
            pl.BlockSpec((1, _TV, _EMBED), lambda b, t: (b, t, 0)),
            pl.BlockSpec((1, 1, 1, _TV), lambda b, t: (b, t, 0, 0)),
            pl.BlockSpec((_EMBED, _EMBED), lambda b, t: (0, 0)),
            pl.BlockSpec((1, _EMBED), lambda b, t: (0, 0)),
        ],
        out_specs=pl.BlockSpec((1, _NH, _TV, _C // 2), lambda b, t: (b, 0, t, 0)),
        out_shape=jax.ShapeDtypeStruct((_BS, _NH, _LV, _C // 2), jnp.int32),
    )(value, maskf, W_value, b_value)


_QT = 256


def _prep_body(q_ref, rp_ref, wox_ref, box_ref, woy_ref, boy_ref,
               wat_ref, bat_ref, idx_ref, wts_ref):
    b = pl.program_id(0)
    q = q_ref[0]
    offx = jnp.dot(q, wox_ref[...], preferred_element_type=jnp.float32) + box_ref[...]
    offy = jnp.dot(q, woy_ref[...], preferred_element_type=jnp.float32) + boy_ref[...]
    logits = jnp.dot(q, wat_ref[...], preferred_element_type=jnp.float32) + bat_ref[...]

    el = jnp.exp(logits)
    hr = lax.broadcasted_iota(jnp.int32, (128, 128), 0) >> 4
    hc = lax.broadcasted_iota(jnp.int32, (128, 128), 1) >> 4
    seg = (hr == hc).astype(jnp.float32)
    aw = el / jnp.dot(el, seg, preferred_element_type=jnp.float32,
                      precision=lax.Precision.HIGHEST)

    col = lax.broadcasted_iota(jnp.int32, (_QT, 128), 1)
    lvl = (col >> 2) & 3
    wvec_i = jnp.right_shift(jnp.int32(64), lvl)
    wvec = wvec_i.astype(jnp.float32)
    lb = (16384 - jnp.right_shift(jnp.int32(16384), 2 * lvl)) // 3

    rowi = lax.broadcasted_iota(jnp.int32, (8, 128), 0)
    lvlj = (lax.broadcasted_iota(jnp.int32, (8, 128), 1) >> 2) & 3
    selx = (rowi == lvlj).astype(jnp.float32)
    sely = (rowi == lvlj + 4).astype(jnp.float32)
    rp = rp_ref[0]
    rxw = jnp.dot(rp, selx * wvec[:1], preferred_element_type=jnp.float32,
                  precision=lax.Precision.HIGHEST)
    ryw = jnp.dot(rp, sely * wvec[:1], preferred_element_type=jnp.float32,
                  precision=lax.Precision.HIGHEST)

    x = rxw + offx - 0.5
    y = ryw + offy - 0.5
    x0f = jnp.floor(x)
    y0f = jnp.floor(y)
    fx = x - x0f
    fy = y - y0f
    x0in = (x0f >= 0.0) & (x0f <= wvec - 1.0)
    x1in = (x0f + 1.0 >= 0.0) & (x0f + 1.0 <= wvec - 1.0)
    y0in = (y0f >= 0.0) & (y0f <= wvec - 1.0)
    y1in = (y0f + 1.0 >= 0.0) & (y0f + 1.0 <= wvec - 1.0)
    x0c = jnp.clip(x0f, 0.0, wvec - 1.0).astype(jnp.int32)
    x1c = jnp.clip(x0f + 1.0, 0.0, wvec - 1.0).astype(jnp.int32)
    y0c = jnp.clip(y0f, 0.0, wvec - 1.0).astype(jnp.int32)
    y1c = jnp.clip(y0f + 1.0, 0.0, wvec - 1.0).astype(jnp.int32)
    gb = lb + (b * _NH + (col >> 4)) * _LV
    r0 = gb + y0c * wvec_i
    r1 = gb + y1c * wvec_i
    gx = 1.0 - fx
    gy = 1.0 - fy
    ias = [r0 + x0c, r1 + x0c, r0 + x1c, r1 + x1c]
    was = [jnp.where(x0in & y0in, gx * gy, 0.0) * aw,
           jnp.where(x0in & y1in, gx * fy, 0.0) * aw,
           jnp.where(x1in & y0in, fx * gy, 0.0) * aw,
           jnp.where(x1in & y1in, fx * fy, 0.0) * aw]
    for c in range(4):
        for h in range(_NH):
            idx_ref[0, c, h] = ias[c][:, h * 16:(h + 1) * 16]
            wts_ref[0, c, h] = was[c][:, h * 16:(h + 1) * 16]


def _prep(query, rp, Wox, box, Woy, boy, W_attn, b_attn):
    return pl.pallas_call(
        _prep_body,
        grid=(_BS, _LQ // _QT),
        in_specs=[
            pl.BlockSpec((1, _QT, _EMBED), lambda b, t: (b, t, 0)),
            pl.BlockSpec((1, _QT, 2 * _NL), lambda b, t: (b, t, 0)),
            pl.BlockSpec((_EMBED, 128), lambda b, t: (0, 0)),
            pl.BlockSpec((1, 128), lambda b, t: (0, 0)),
            pl.BlockSpec((_EMBED, 128), lambda b, t: (0, 0)),
            pl.BlockSpec((1, 128), lambda b, t: (0, 0)),
            pl.BlockSpec((_EMBED, 128), lambda b, t: (0, 0)),
            pl.BlockSpec((1, 128), lambda b, t: (0, 0)),
        ],
        out_specs=[
            pl.BlockSpec((1, 4, _NH, _QT, 16), lambda b, t: (b, 0, 0, t, 0)),
            pl.BlockSpec((1, 4, _NH, _QT, 16), lambda b, t: (b, 0, 0, t, 0)),
        ],
        out_shape=[
            jax.ShapeDtypeStruct((_BS, 4, _NH, _LQ, 16), jnp.int32),
            jax.ShapeDtypeStruct((_BS, 4, _NH, _LQ, 16), jnp.float32),
        ],
    )(query, rp, Wox, box, Woy, boy, W_attn, b_attn)


_CQ = 16
_NCHUNK = _LQ // _CQ
_NROW = _CQ * _NCORN


_NB = _NROW // 128


def _sc_sample(table, idx4, wts4):
    info = plsc.get_sparse_core_info()
    nc = info.num_cores
    mesh = plsc.VectorSubcoreMesh(core_axis_name="c", subcore_axis_name="s")

    @functools.partial(
        pl.kernel,
        out_type=jax.ShapeDtypeStruct((_BS * _NH, _LQ, _C), jnp.float32),
        mesh=mesh,
        compiler_params=pltpu.CompilerParams(needs_layout_passes=False,
                                             use_tc_tiling_on_sc=False),
        scratch_types=[
            pltpu.VMEM((_NB, 128), jnp.int32),
            pltpu.VMEM((_NB, 128), jnp.int32),
            pltpu.VMEM((4 * _CQ, 16), jnp.float32),
            pltpu.VMEM((4 * _CQ, 16), jnp.float32),
            pltpu.VMEM((_NROW, _C // 2), jnp.int32),
            pltpu.VMEM((_NROW, _C // 2), jnp.int32),
            pltpu.VMEM((_CQ, _C), jnp.float32),
            pltpu.SemaphoreType.DMA,
            pltpu.SemaphoreType.DMA,
            pltpu.SemaphoreType.DMA,
        ],
    )
    def run(table_h, idx_h, wts_h, out_h, idx0, idx1, wts0, wts1,
            rows0, rows1, out_v, gs0, gs1, lsem):
        wid = lax.axis_index("s") * nc + lax.axis_index("c")
        bb = wid >> 3
        hh = wid & 7
        col0 = lax.iota(jnp.int32, 16)
        col1 = col0 + 16
        lpc = [jnp.full((16,), lp, jnp.int32) for lp in range(16)]

        def load_iw(n, idxv, wtsv):
            descs = []
            for c in range(4):
                descs.append(pltpu.async_copy(
                    idx_h.at[bb, c, hh, n], idxv.at[pl.ds(c * 2, 2)], lsem))
                descs.append(pltpu.async_copy(
                    wts_h.at[bb, c, hh, pl.ds(n * _CQ, _CQ)],
                    wtsv.at[pl.ds(c * _CQ, _CQ)], lsem))
            for d in descs:
                d.wait()

        def gather(idxv, rowsv, sem):
            for k in range(_NB):
                pltpu.async_copy(table_h.at[idxv.at[k]],
                                 rowsv.at[pl.ds(k * 128, 128)], sem)

        def drain(rowsv, sem):
            pltpu.make_async_copy(table_h.at[pl.ds(0, _NROW)], rowsv, sem).wait()

        def compute(wtsv, rowsv, n):
            def qloop(qi, c2):
                qsp = jnp.full((16,), qi, jnp.int32)
                base = jnp.full((16,), qi * _CQ, jnp.int32)
                acc0 = None
                acc1 = None
                for c in range(4):
                    wrow = qsp + c * _CQ
                    for lp in range(16):
                        rsp = base + (c * 256 + lp)
                        w = plsc.load_gather(wtsv, [wrow, lpc[lp]])
                        ri = plsc.load_gather(rowsv, [rsp, col0])
                        bf = plsc.bitcast(ri, jnp.bfloat16)
                        re, ro = plsc.unpack(bf, format=plsc.PackFormat.INTERLEAVED)
                        if acc0 is None:
                            acc0 = w * re
                            acc1 = w * ro
                        else:
                            acc0 = acc0 + w * re
                            acc1 = acc1 + w * ro
                plsc.store_scatter(out_v, [qsp, col0], acc0)
                plsc.store_scatter(out_v, [qsp, col1], acc1)
                return c2

            lax.fori_loop(0, _CQ, qloop, 0)
            pltpu.sync_copy(out_v, out_h.at[wid, pl.ds(n * _CQ, _CQ)])

        load_iw(0, idx0, wts0)
        gather(idx0, rows0, gs0)

        def body(i, carry):
            n0 = 2 * i
            load_iw(n0 + 1, idx1, wts1)
            gather(idx1, rows1, gs1)
            drain(rows0, gs0)
            compute(wts0, rows0, n0)

            @pl.when(i < _NCHUNK // 2 - 1)
            def _():
                load_iw(n0 + 2, idx0, wts0)
                gather(idx0, rows0, gs0)

            drain(rows1, gs1)
            compute(wts1, rows1, n0 + 1)
            return carry

        lax.fori_loop(0, _NCHUNK // 2, body, 0)

    return run(table, idx4, wts4)


_QTC = 512


def _outproj_body(s_ref, w_ref, b_ref, o_ref):
    parts = [s_ref[0, h] for h in range(_NH)]
    x = jnp.concatenate(parts, axis=1)
    o_ref[0] = jnp.dot(x, w_ref[...], preferred_element_type=jnp.float32) + b_ref[...]


def _outproj(sampled, W_out, b_out):
    return pl.pallas_call(
        _outproj_body,
        grid=(_BS, _LQ // _QTC),
        in_specs=[
            pl.BlockSpec((1, _NH, _QTC, _C), lambda b, t: (b, 0, t, 0)),
            pl.BlockSpec((_EMBED, _EMBED), lambda b, t: (0, 0)),
            pl.BlockSpec((1, _EMBED), lambda b, t: (0, 0)),
        ],
        out_specs=pl.BlockSpec((1, _QTC, _EMBED), lambda b, t: (b, t, 0)),
        out_shape=jax.ShapeDtypeStruct((_BS, _LQ, _EMBED), jnp.float32),
    )(sampled, W_out, b_out)


def kernel(query, ref_points, value, pad_mask, W_value, b_value, W_off, b_off,
           W_attn, b_attn, W_out, b_out):
    maskf = pad_mask.astype(jnp.float32).reshape(_BS, _LV // _TV, 1, _TV)
    table = _vproj(value, maskf, W_value, b_value.reshape(1, _EMBED))
    table = table.reshape(_BS * _NH * _LV, _C // 2)

    Wo = W_off.reshape(_EMBED, _NH * _NL * _NP, 2)
    bo = b_off.reshape(_NH * _NL * _NP, 2)
    rp = jnp.concatenate([ref_points[..., 0], ref_points[..., 1]], axis=-1)
    idx, wts = _prep(query, rp,
                     Wo[..., 0], bo[:, 0].reshape(1, -1),
                     Wo[..., 1], bo[:, 1].reshape(1, -1),
                     W_attn, b_attn.reshape(1, -1))

    idxr = idx.reshape(_BS, 4, _NH, _NCHUNK, 2, 128)
    sampled = _sc_sample(table, idxr, wts)
    sampled = sampled.reshape(_BS, _NH, _LQ, _C)
    return _outproj(sampled, W_out, b_out.reshape(1, _EMBED))


# speedup vs baseline: 1.4774x; 1.0136x over previous
import functools
import jax
import jax.numpy as jnp
from jax import lax
from jax.experimental import pallas as pl
from jax.experimental.pallas import tpu as pltpu
from jax.experimental.pallas import tpu_sc as plsc

_SPATIAL = ((64, 64), (32, 32), (16, 16), (8, 8))
_LVL_BASE = (0, 4096, 5120, 5376)
_EMBED = 256
_NL = 4
_NH = 8
_NP = 4
_BS = 4
_LQ = 1024
_LV = 5440
_C = 32
_NCORN = _NL * _NP * 4

_TV = 680


def _vproj_body(val_ref, msk_ref, w_ref, b_ref, out_ref):
    x = val_ref[0]
    v = jnp.dot(x, w_ref[...], preferred_element_type=jnp.float32)
    v = ((v + b_ref[...]) * msk_ref[0, 0, 0][:, None]).astype(jnp.bfloat16)
    for h in range(_NH):
        lo = jax.lax.bitcast_convert_type(
            v[:, h * _C:h * _C + 16], jnp.int16).astype(jnp.int32) & 0xFFFF
        hi = jax.lax.bitcast_convert_type(
            v[:, h * _C + 16:(h + 1) * _C], jnp.int16).astype(jnp.int32)
        out_ref[0, h] = lo | (hi << 16)


def _vproj(value, maskf, W_value, b_value):
    return pl.pallas_call(
        _vproj_body,
        grid=(_BS, _LV // _TV),
        in_specs=[
            pl.BlockSpec((1, _TV, _EMBED), lambda b, t: (b, t, 0)),
            pl.BlockSpec((1, 1, 1, _TV), lambda b, t: (b, t, 0, 0)),
            pl.BlockSpec((_EMBED, _EMBED), lambda b, t: (0, 0)),
            pl.BlockSpec((1, _EMBED), lambda b, t: (0, 0)),
        ],
        out_specs=pl.BlockSpec((1, _NH, _TV, _C // 2), lambda b, t: (b, 0, t, 0)),
        out_shape=jax.ShapeDtypeStruct((_BS, _NH, _LV, _C // 2), jnp.int32),
    )(value, maskf, W_value, b_value)


_QT = 256


def _prep_body(q_ref, rp_ref, wox_ref, box_ref, woy_ref, boy_ref,
               wat_ref, bat_ref, idx_ref, wts_ref):
    b = pl.program_id(0)
    q = q_ref[0]
    offx = jnp.dot(q, wox_ref[...], preferred_element_type=jnp.float32) + box_ref[...]
    offy = jnp.dot(q, woy_ref[...], preferred_element_type=jnp.float32) + boy_ref[...]
    logits = jnp.dot(q, wat_ref[...], preferred_element_type=jnp.float32) + bat_ref[...]

    el = jnp.exp(logits)
    hr = lax.broadcasted_iota(jnp.int32, (128, 128), 0) >> 4
    hc = lax.broadcasted_iota(jnp.int32, (128, 128), 1) >> 4
    seg = (hr == hc).astype(jnp.float32)
    aw = el / jnp.dot(el, seg, preferred_element_type=jnp.float32,
                      precision=lax.Precision.HIGHEST)

    col = lax.broadcasted_iota(jnp.int32, (_QT, 128), 1)
    lvl = (col >> 2) & 3
    wvec_i = jnp.right_shift(jnp.int32(64), lvl)
    wvec = wvec_i.astype(jnp.float32)
    lb = (16384 - jnp.right_shift(jnp.int32(16384), 2 * lvl)) // 3

    rowi = lax.broadcasted_iota(jnp.int32, (8, 128), 0)
    lvlj = (lax.broadcasted_iota(jnp.int32, (8, 128), 1) >> 2) & 3
    selx = (rowi == lvlj).astype(jnp.float32)
    sely = (rowi == lvlj + 4).astype(jnp.float32)
    rp = rp_ref[0]
    rxw = jnp.dot(rp, selx * wvec[:1], preferred_element_type=jnp.float32,
                  precision=lax.Precision.HIGHEST)
    ryw = jnp.dot(rp, sely * wvec[:1], preferred_element_type=jnp.float32,
                  precision=lax.Precision.HIGHEST)

    x = rxw + offx - 0.5
    y = ryw + offy - 0.5
    x0f = jnp.floor(x)
    y0f = jnp.floor(y)
    fx = x - x0f
    fy = y - y0f
    x0in = (x0f >= 0.0) & (x0f <= wvec - 1.0)
    x1in = (x0f + 1.0 >= 0.0) & (x0f + 1.0 <= wvec - 1.0)
    y0in = (y0f >= 0.0) & (y0f <= wvec - 1.0)
    y1in = (y0f + 1.0 >= 0.0) & (y0f + 1.0 <= wvec - 1.0)
    x0c = jnp.clip(x0f, 0.0, wvec - 1.0).astype(jnp.int32)
    x1c = jnp.clip(x0f + 1.0, 0.0, wvec - 1.0).astype(jnp.int32)
    y0c = jnp.clip(y0f, 0.0, wvec - 1.0).astype(jnp.int32)
    y1c = jnp.clip(y0f + 1.0, 0.0, wvec - 1.0).astype(jnp.int32)
    gb = lb + (b * _NH + (col >> 4)) * _LV
    r0 = gb + y0c * wvec_i
    r1 = gb + y1c * wvec_i
    gx = 1.0 - fx
    gy = 1.0 - fy
    ias = [r0 + x0c, r1 + x0c, r0 + x1c, r1 + x1c]
    was = [jnp.where(x0in & y0in, gx * gy, 0.0) * aw,
           jnp.where(x0in & y1in, gx * fy, 0.0) * aw,
           jnp.where(x1in & y0in, fx * gy, 0.0) * aw,
           jnp.where(x1in & y1in, fx * fy, 0.0) * aw]
    for c in range(4):
        for h in range(_NH):
            idx_ref[0, c, h] = ias[c][:, h * 16:(h + 1) * 16]
            wts_ref[0, c, h] = was[c][:, h * 16:(h + 1) * 16]


def _prep(query, rp, Wox, box, Woy, boy, W_attn, b_attn):
    return pl.pallas_call(
        _prep_body,
        grid=(_BS, _LQ // _QT),
        in_specs=[
            pl.BlockSpec((1, _QT, _EMBED), lambda b, t: (b, t, 0)),
            pl.BlockSpec((1, _QT, 2 * _NL), lambda b, t: (b, t, 0)),
            pl.BlockSpec((_EMBED, 128), lambda b, t: (0, 0)),
            pl.BlockSpec((1, 128), lambda b, t: (0, 0)),
            pl.BlockSpec((_EMBED, 128), lambda b, t: (0, 0)),
            pl.BlockSpec((1, 128), lambda b, t: (0, 0)),
            pl.BlockSpec((_EMBED, 128), lambda b, t: (0, 0)),
            pl.BlockSpec((1, 128), lambda b, t: (0, 0)),
        ],
        out_specs=[
            pl.BlockSpec((1, 4, _NH, _QT, 16), lambda b, t: (b, 0, 0, t, 0)),
            pl.BlockSpec((1, 4, _NH, _QT, 16), lambda b, t: (b, 0, 0, t, 0)),
        ],
        out_shape=[
            jax.ShapeDtypeStruct((_BS, 4, _NH, _LQ, 16), jnp.int32),
            jax.ShapeDtypeStruct((_BS, 4, _NH, _LQ, 16), jnp.float32),
        ],
    )(query, rp, Wox, box, Woy, boy, W_attn, b_attn)


_CQ = 32
_NCHUNK = _LQ // _CQ
_NROW = _CQ * _NCORN
_NB = _NROW // 128
_IB = _CQ * 16 // 128


def _sc_sample(table, idx4, wts4):
    info = plsc.get_sparse_core_info()
    nc = info.num_cores
    mesh = plsc.VectorSubcoreMesh(core_axis_name="c", subcore_axis_name="s")

    @functools.partial(
        pl.kernel,
        out_type=jax.ShapeDtypeStruct((_BS * _NH, _LQ, _C), jnp.float32),
        mesh=mesh,
        compiler_params=pltpu.CompilerParams(needs_layout_passes=False,
                                             use_tc_tiling_on_sc=False),
        scratch_types=[
            pltpu.VMEM((_NB, 128), jnp.int32),
            pltpu.VMEM((_NB, 128), jnp.int32),
            pltpu.VMEM((4 * _CQ, 16), jnp.float32),
            pltpu.VMEM((4 * _CQ, 16), jnp.float32),
            pltpu.VMEM((_NROW, _C // 2), jnp.int32),
            pltpu.VMEM((_NROW, _C // 2), jnp.int32),
            pltpu.VMEM((_CQ, _C), jnp.float32),
            pltpu.SemaphoreType.DMA,
            pltpu.SemaphoreType.DMA,
            pltpu.SemaphoreType.DMA,
        ],
    )
    def run(table_h, idx_h, wts_h, out_h, idx0, idx1, wts0, wts1,
            rows0, rows1, out_v, gs0, gs1, lsem):
        wid = lax.axis_index("s") * nc + lax.axis_index("c")
        bb = wid >> 3
        hh = wid & 7
        col0 = lax.iota(jnp.int32, 16)
        col1 = col0 + 16
        lpc = [jnp.full((16,), lp, jnp.int32) for lp in range(16)]

        def load_iw(n, idxv, wtsv):
            descs = []
            for c in range(4):
                descs.append(pltpu.async_copy(
                    idx_h.at[bb, c, hh, n], idxv.at[pl.ds(c * _IB, _IB)],
                    lsem))
                descs.append(pltpu.async_copy(
                    wts_h.at[bb, c, hh, pl.ds(n * _CQ, _CQ)],
                    wtsv.at[pl.ds(c * _CQ, _CQ)], lsem))
            for d in descs:
                d.wait()

        def gather(idxv, rowsv, sem):
            for k in range(_NB):
                pltpu.async_copy(table_h.at[idxv.at[k]],
                                 rowsv.at[pl.ds(k * 128, 128)], sem)

        def drain(rowsv, sem):
            pltpu.make_async_copy(table_h.at[pl.ds(0, _NROW)], rowsv, sem).wait()

        def compute(wtsv, rowsv, n):
            def qloop(qi, c2):
                qsp = jnp.full((16,), qi, jnp.int32)
                base = jnp.full((16,), qi * 16, jnp.int32)
                acc0 = None
                acc1 = None
                for c in range(4):
                    wrow = qsp + c * _CQ
                    wv = plsc.load_gather(wtsv, [wrow, col0])
                    for lp in range(16):
                        rsp = base + (c * _CQ * 16 + lp)
                        w = wv.at[lpc[lp]].get(mode="promise_in_bounds")
                        ri = plsc.load_gather(rowsv, [rsp, col0])
                        bf = plsc.bitcast(ri, jnp.bfloat16)
                        re, ro = plsc.unpack(bf, format=plsc.PackFormat.INTERLEAVED)
                        if acc0 is None:
                            acc0 = w * re
                            acc1 = w * ro
                        else:
                            acc0 = acc0 + w * re
                            acc1 = acc1 + w * ro
                plsc.store_scatter(out_v, [qsp, col0], acc0)
                plsc.store_scatter(out_v, [qsp, col1], acc1)
                return c2

            lax.fori_loop(0, _CQ, qloop, 0)
            pltpu.sync_copy(out_v, out_h.at[wid, pl.ds(n * _CQ, _CQ)])

        load_iw(0, idx0, wts0)
        gather(idx0, rows0, gs0)

        def body(i, carry):
            n0 = 2 * i
            load_iw(n0 + 1, idx1, wts1)
            gather(idx1, rows1, gs1)
            drain(rows0, gs0)
            compute(wts0, rows0, n0)

            @pl.when(i < _NCHUNK // 2 - 1)
            def _():
                load_iw(n0 + 2, idx0, wts0)
                gather(idx0, rows0, gs0)

            drain(rows1, gs1)
            compute(wts1, rows1, n0 + 1)
            return carry

        lax.fori_loop(0, _NCHUNK // 2, body, 0)

    return run(table, idx4, wts4)


_QTC = 512


def _outproj_body(s_ref, w_ref, b_ref, o_ref):
    parts = [s_ref[0, h] for h in range(_NH)]
    x = jnp.concatenate(parts, axis=1)
    o_ref[0] = jnp.dot(x, w_ref[...], preferred_element_type=jnp.float32) + b_ref[...]


def _outproj(sampled, W_out, b_out):
    return pl.pallas_call(
        _outproj_body,
        grid=(_BS, _LQ // _QTC),
        in_specs=[
            pl.BlockSpec((1, _NH, _QTC, _C), lambda b, t: (b, 0, t, 0)),
            pl.BlockSpec((_EMBED, _EMBED), lambda b, t: (0, 0)),
            pl.BlockSpec((1, _EMBED), lambda b, t: (0, 0)),
        ],
        out_specs=pl.BlockSpec((1, _QTC, _EMBED), lambda b, t: (b, t, 0)),
        out_shape=jax.ShapeDtypeStruct((_BS, _LQ, _EMBED), jnp.float32),
    )(sampled, W_out, b_out)


def kernel(query, ref_points, value, pad_mask, W_value, b_value, W_off, b_off,
           W_attn, b_attn, W_out, b_out):
    maskf = pad_mask.astype(jnp.float32).reshape(_BS, _LV // _TV, 1, _TV)
    table = _vproj(value, maskf, W_value, b_value.reshape(1, _EMBED))
    table = table.reshape(_BS * _NH * _LV, _C // 2)

    Wo = W_off.reshape(_EMBED, _NH * _NL * _NP, 2)
    bo = b_off.reshape(_NH * _NL * _NP, 2)
    rp = jnp.concatenate([ref_points[..., 0], ref_points[..., 1]], axis=-1)
    idx, wts = _prep(query, rp,
                     Wo[..., 0], bo[:, 0].reshape(1, -1),
                     Wo[..., 1], bo[:, 1].reshape(1, -1),
                     W_attn, b_attn.reshape(1, -1))

    idxr = idx.reshape(_BS, 4, _NH, _NCHUNK, _IB, 128)
    sampled = _sc_sample(table, idxr, wts)
    sampled = sampled.reshape(_BS, _NH, _LQ, _C)
    return _outproj(sampled, W_out, b_out.reshape(1, _EMBED))
